# Initial kernel scaffold; baseline (speedup 1.0000x reference)
import jax
import jax.numpy as jnp
from jax.experimental import pallas as pl


def kernel(state, next_state, action, w1p, b1p, w2p, b2p, w3p, b3p, w1i, b1i, w2i, b2i, w3i, b3i):
    raise NotImplementedError("write your pallas kernel here")


# speedup vs baseline: 1.2772x; 1.2772x over previous
import jax
import jax.numpy as jnp
from jax.experimental import pallas as pl
from jax.experimental.pallas import tpu as pltpu


def _round_up(x, m):
    return ((x + m - 1) // m) * m


def _icm_kernel(
    state_ref, next_state_ref, action_ref,
    w1ps_ref, w1pa_ref, b1p_ref, w2p_ref, b2p_ref, w3p_ref, b3p_ref,
    w1is_ref, w1in_ref, b1i_ref, w2i_ref, b2i_ref, w3i_ref, b3i_ref,
    ns_out_ref, ap_out_ref,
):
    s = state_ref[...].astype(jnp.bfloat16)
    ns = next_state_ref[...].astype(jnp.bfloat16)
    a = action_ref[...].astype(jnp.bfloat16)

    h = (jnp.dot(s, w1ps_ref[...], preferred_element_type=jnp.float32)
         + jnp.dot(a, w1pa_ref[...], preferred_element_type=jnp.float32)
         + b1p_ref[...])
    h = jnp.maximum(h, 0.0).astype(jnp.bfloat16)
    h = (jnp.dot(h, w2p_ref[...], preferred_element_type=jnp.float32)
         + b2p_ref[...])
    h = jnp.maximum(h, 0.0).astype(jnp.bfloat16)
    ns_out_ref[...] = (
        jnp.dot(h, w3p_ref[...], preferred_element_type=jnp.float32)
        + b3p_ref[...])

    g = (jnp.dot(s, w1is_ref[...], preferred_element_type=jnp.float32)
         + jnp.dot(ns, w1in_ref[...], preferred_element_type=jnp.float32)
         + b1i_ref[...])
    g = jnp.maximum(g, 0.0).astype(jnp.bfloat16)
    g = (jnp.dot(g, w2i_ref[...], preferred_element_type=jnp.float32)
         + b2i_ref[...])
    g = jnp.maximum(g, 0.0).astype(jnp.bfloat16)
    ap_out_ref[...] = (
        jnp.dot(g, w3i_ref[...], preferred_element_type=jnp.float32)
        + b3i_ref[...])


def kernel(state, next_state, action,
           w1p, b1p, w2p, b2p, w3p, b3p,
           w1i, b1i, w2i, b2i, w3i, b3i,
           *, tile_b=512):
    B, S = state.shape
    A = action.shape[1]

    bf16 = jnp.bfloat16
    param_arrays = [
        w1p[:S].astype(bf16), w1p[S:].astype(bf16), b1p,
        w2p.astype(bf16), b2p, w3p.astype(bf16), b3p,
        w1i[:S].astype(bf16), w1i[S:].astype(bf16), b1i,
        w2i.astype(bf16), b2i, w3i.astype(bf16), b3i,
    ]

    tile_b = min(tile_b, _round_up(B, 8))
    b_pad = _round_up(B, tile_b)
    if b_pad != B:
        pad = ((0, b_pad - B), (0, 0))
        state = jnp.pad(state, pad)
        next_state = jnp.pad(next_state, pad)
        action = jnp.pad(action, pad)
    grid = (b_pad // tile_b,)

    def batch_spec(n):
        return pl.BlockSpec((tile_b, n), lambda i: (i, 0))

    def param_spec(shape):
        return pl.BlockSpec(shape, lambda i: (0, 0))

    in_specs = ([batch_spec(S), batch_spec(S), batch_spec(A)]
                + [param_spec(tuple(p.shape)) for p in param_arrays])

    H1 = w1p.shape[1]
    H2 = w2p.shape[1]
    flops = 2 * b_pad * ((S + A) * H1 + 2 * S * H1 + 2 * H1 * H2
                         + H2 * S + H2 * A)
    bytes_accessed = (4 * b_pad * (2 * S + A + S + A)
                      + sum(int(p.size) * p.dtype.itemsize
                            for p in param_arrays))

    ns_pred, a_pred = pl.pallas_call(
        _icm_kernel,
        out_shape=(jax.ShapeDtypeStruct((b_pad, S), jnp.float32),
                   jax.ShapeDtypeStruct((b_pad, A), jnp.float32)),
        grid=grid,
        in_specs=in_specs,
        out_specs=(pl.BlockSpec((tile_b, S), lambda i: (i, 0)),
                   pl.BlockSpec((tile_b, A), lambda i: (i, 0))),
        compiler_params=pltpu.CompilerParams(
            dimension_semantics=("parallel",)),
        cost_estimate=pl.CostEstimate(
            flops=flops, transcendentals=0, bytes_accessed=bytes_accessed),
    )(state, next_state, action, *param_arrays)

    return ns_pred[:B], a_pred[:B]


# speedup vs baseline: 1.4283x; 1.1183x over previous
import jax
import jax.numpy as jnp
from jax.experimental import pallas as pl
from jax.experimental.pallas import tpu as pltpu


def _round_up(x, m):
    return ((x + m - 1) // m) * m


def _icm_kernel(
    state_ref, next_state_ref, action_ref,
    w1ps_ref, w1pa_ref, b1p_ref, w2p_ref, b2p_ref, w3p_ref, b3p_ref,
    w1is_ref, w1in_ref, b1i_ref, w2i_ref, b2i_ref, w3i_ref, b3i_ref,
    ns_out_ref, ap_out_ref,
):
    s = state_ref[...].astype(jnp.bfloat16)
    ns = next_state_ref[...].astype(jnp.bfloat16)
    a = action_ref[...].astype(jnp.bfloat16)

    h = (jnp.dot(s, w1ps_ref[...], preferred_element_type=jnp.float32)
         + jnp.dot(a, w1pa_ref[...], preferred_element_type=jnp.float32)
         + b1p_ref[...])
    h = jnp.maximum(h, 0.0).astype(jnp.bfloat16)
    h = (jnp.dot(h, w2p_ref[...], preferred_element_type=jnp.float32)
         + b2p_ref[...])
    h = jnp.maximum(h, 0.0).astype(jnp.bfloat16)
    ns_out_ref[...] = (
        jnp.dot(h, w3p_ref[...], preferred_element_type=jnp.float32)
        + b3p_ref[...])

    g = (jnp.dot(s, w1is_ref[...], preferred_element_type=jnp.float32)
         + jnp.dot(ns, w1in_ref[...], preferred_element_type=jnp.float32)
         + b1i_ref[...])
    g = jnp.maximum(g, 0.0).astype(jnp.bfloat16)
    g = (jnp.dot(g, w2i_ref[...], preferred_element_type=jnp.float32)
         + b2i_ref[...])
    g = jnp.maximum(g, 0.0).astype(jnp.bfloat16)
    ap_out_ref[...] = (
        jnp.dot(g, w3i_ref[...], preferred_element_type=jnp.float32)
        + b3i_ref[...])


def kernel(state, next_state, action,
           w1p, b1p, w2p, b2p, w3p, b3p,
           w1i, b1i, w2i, b2i, w3i, b3i,
           *, tile_b=1024):
    B, S = state.shape
    A = action.shape[1]

    bf16 = jnp.bfloat16
    param_arrays = [
        w1p[:S].astype(bf16), w1p[S:].astype(bf16), b1p,
        w2p.astype(bf16), b2p, w3p.astype(bf16), b3p,
        w1i[:S].astype(bf16), w1i[S:].astype(bf16), b1i,
        w2i.astype(bf16), b2i, w3i.astype(bf16), b3i,
    ]

    tile_b = min(tile_b, _round_up(B, 8))
    b_pad = _round_up(B, tile_b)
    if b_pad != B:
        pad = ((0, b_pad - B), (0, 0))
        state = jnp.pad(state, pad)
        next_state = jnp.pad(next_state, pad)
        action = jnp.pad(action, pad)
    grid = (b_pad // tile_b,)

    def batch_spec(n):
        return pl.BlockSpec((tile_b, n), lambda i: (i, 0))

    def param_spec(shape):
        return pl.BlockSpec(shape, lambda i: (0, 0))

    in_specs = ([batch_spec(S), batch_spec(S), batch_spec(A)]
                + [param_spec(tuple(p.shape)) for p in param_arrays])

    H1 = w1p.shape[1]
    H2 = w2p.shape[1]
    flops = 2 * b_pad * ((S + A) * H1 + 2 * S * H1 + 2 * H1 * H2
                         + H2 * S + H2 * A)
    bytes_accessed = (4 * b_pad * (2 * S + A + S + A)
                      + sum(int(p.size) * p.dtype.itemsize
                            for p in param_arrays))

    ns_pred, a_pred = pl.pallas_call(
        _icm_kernel,
        out_shape=(jax.ShapeDtypeStruct((b_pad, S), jnp.float32),
                   jax.ShapeDtypeStruct((b_pad, A), jnp.float32)),
        grid=grid,
        in_specs=in_specs,
        out_specs=(pl.BlockSpec((tile_b, S), lambda i: (i, 0)),
                   pl.BlockSpec((tile_b, A), lambda i: (i, 0))),
        compiler_params=pltpu.CompilerParams(
            dimension_semantics=("parallel",)),
        cost_estimate=pl.CostEstimate(
            flops=flops, transcendentals=0, bytes_accessed=bytes_accessed),
    )(state, next_state, action, *param_arrays)

    return ns_pred[:B], a_pred[:B]


# speedup vs baseline: 1.4593x; 1.0217x over previous
import jax
import jax.numpy as jnp
from jax.experimental import pallas as pl
from jax.experimental.pallas import tpu as pltpu


def _round_up(x, m):
    return ((x + m - 1) // m) * m


def _icm_kernel(
    state_ref, next_state_ref, action_ref,
    w1ps_ref, w1pa_ref, b1p_ref, w2p_ref, b2p_ref, w3p_ref, b3p_ref,
    w1is_ref, w1in_ref, b1i_ref, w2i_ref, b2i_ref, w3i_ref, b3i_ref,
    ns_out_ref, ap_out_ref,
):
    s = state_ref[...].astype(jnp.bfloat16)
    ns = next_state_ref[...].astype(jnp.bfloat16)
    a = action_ref[...].astype(jnp.bfloat16)

    h = (jnp.dot(s, w1ps_ref[...], preferred_element_type=jnp.float32)
         + jnp.dot(a, w1pa_ref[...], preferred_element_type=jnp.float32)
         + b1p_ref[...])
    h = jnp.maximum(h, 0.0).astype(jnp.bfloat16)
    h = (jnp.dot(h, w2p_ref[...], preferred_element_type=jnp.float32)
         + b2p_ref[...])
    h = jnp.maximum(h, 0.0).astype(jnp.bfloat16)
    ns_out_ref[...] = (
        jnp.dot(h, w3p_ref[...], preferred_element_type=jnp.float32)
        + b3p_ref[...])

    g = (jnp.dot(s, w1is_ref[...], preferred_element_type=jnp.float32)
         + jnp.dot(ns, w1in_ref[...], preferred_element_type=jnp.float32)
         + b1i_ref[...])
    g = jnp.maximum(g, 0.0).astype(jnp.bfloat16)
    g = (jnp.dot(g, w2i_ref[...], preferred_element_type=jnp.float32)
         + b2i_ref[...])
    g = jnp.maximum(g, 0.0).astype(jnp.bfloat16)
    ap_out_ref[...] = (
        jnp.dot(g, w3i_ref[...], preferred_element_type=jnp.float32)
        + b3i_ref[...])


def kernel(state, next_state, action,
           w1p, b1p, w2p, b2p, w3p, b3p,
           w1i, b1i, w2i, b2i, w3i, b3i,
           *, tile_b=2048):
    B, S = state.shape
    A = action.shape[1]

    bf16 = jnp.bfloat16
    param_arrays = [
        w1p[:S].astype(bf16), w1p[S:].astype(bf16), b1p,
        w2p.astype(bf16), b2p, w3p.astype(bf16), b3p,
        w1i[:S].astype(bf16), w1i[S:].astype(bf16), b1i,
        w2i.astype(bf16), b2i, w3i.astype(bf16), b3i,
    ]

    tile_b = min(tile_b, _round_up(B, 8))
    b_pad = _round_up(B, tile_b)
    if b_pad != B:
        pad = ((0, b_pad - B), (0, 0))
        state = jnp.pad(state, pad)
        next_state = jnp.pad(next_state, pad)
        action = jnp.pad(action, pad)
    grid = (b_pad // tile_b,)

    def batch_spec(n):
        return pl.BlockSpec((tile_b, n), lambda i: (i, 0))

    def param_spec(shape):
        return pl.BlockSpec(shape, lambda i: (0, 0))

    in_specs = ([batch_spec(S), batch_spec(S), batch_spec(A)]
                + [param_spec(tuple(p.shape)) for p in param_arrays])

    H1 = w1p.shape[1]
    H2 = w2p.shape[1]
    flops = 2 * b_pad * ((S + A) * H1 + 2 * S * H1 + 2 * H1 * H2
                         + H2 * S + H2 * A)
    bytes_accessed = (4 * b_pad * (2 * S + A + S + A)
                      + sum(int(p.size) * p.dtype.itemsize
                            for p in param_arrays))

    ns_pred, a_pred = pl.pallas_call(
        _icm_kernel,
        out_shape=(jax.ShapeDtypeStruct((b_pad, S), jnp.float32),
                   jax.ShapeDtypeStruct((b_pad, A), jnp.float32)),
        grid=grid,
        in_specs=in_specs,
        out_specs=(pl.BlockSpec((tile_b, S), lambda i: (i, 0)),
                   pl.BlockSpec((tile_b, A), lambda i: (i, 0))),
        compiler_params=pltpu.CompilerParams(
            dimension_semantics=("parallel",)),
        cost_estimate=pl.CostEstimate(
            flops=flops, transcendentals=0, bytes_accessed=bytes_accessed),
    )(state, next_state, action, *param_arrays)

    return ns_pred[:B], a_pred[:B]


# speedup vs baseline: 2.0550x; 1.4082x over previous
import jax
import jax.numpy as jnp
from jax.experimental import pallas as pl
from jax.experimental.pallas import tpu as pltpu


def _round_up(x, m):
    return ((x + m - 1) // m) * m


def _make_icm_kernel(S):
    def _icm_kernel(
        state_ref, next_state_ref, action_ref,
        w1p_ref, b1p_ref, w2p_ref, b2p_ref, w3p_ref, b3p_ref,
        w1i_ref, b1i_ref, w2i_ref, b2i_ref, w3i_ref, b3i_ref,
        ns_out_ref, ap_out_ref,
    ):
        bf16 = jnp.bfloat16
        s = state_ref[...].astype(bf16)
        ns = next_state_ref[...].astype(bf16)
        a = action_ref[...].astype(bf16)

        h = (jnp.dot(s, w1p_ref[:S].astype(bf16),
                     preferred_element_type=jnp.float32)
             + jnp.dot(a, w1p_ref[S:].astype(bf16),
                       preferred_element_type=jnp.float32)
             + b1p_ref[...])
        h = jnp.maximum(h, 0.0).astype(bf16)
        h = (jnp.dot(h, w2p_ref[...].astype(bf16),
                     preferred_element_type=jnp.float32)
             + b2p_ref[...])
        h = jnp.maximum(h, 0.0).astype(bf16)
        ns_out_ref[...] = (
            jnp.dot(h, w3p_ref[...].astype(bf16),
                    preferred_element_type=jnp.float32)
            + b3p_ref[...])

        g = (jnp.dot(s, w1i_ref[:S].astype(bf16),
                     preferred_element_type=jnp.float32)
             + jnp.dot(ns, w1i_ref[S:].astype(bf16),
                       preferred_element_type=jnp.float32)
             + b1i_ref[...])
        g = jnp.maximum(g, 0.0).astype(bf16)
        g = (jnp.dot(g, w2i_ref[...].astype(bf16),
                     preferred_element_type=jnp.float32)
             + b2i_ref[...])
        g = jnp.maximum(g, 0.0).astype(bf16)
        ap_out_ref[...] = (
            jnp.dot(g, w3i_ref[...].astype(bf16),
                    preferred_element_type=jnp.float32)
            + b3i_ref[...])

    return _icm_kernel


def kernel(state, next_state, action,
           w1p, b1p, w2p, b2p, w3p, b3p,
           w1i, b1i, w2i, b2i, w3i, b3i,
           *, tile_b=2048):
    B, S = state.shape
    A = action.shape[1]

    tile_b = min(tile_b, _round_up(B, 8))
    b_pad = _round_up(B, tile_b)
    if b_pad != B:
        pad = ((0, b_pad - B), (0, 0))
        state = jnp.pad(state, pad)
        next_state = jnp.pad(next_state, pad)
        action = jnp.pad(action, pad)
    grid = (b_pad // tile_b,)

    param_arrays = [w1p, b1p, w2p, b2p, w3p, b3p,
                    w1i, b1i, w2i, b2i, w3i, b3i]

    def batch_spec(n):
        return pl.BlockSpec((tile_b, n), lambda i: (i, 0))

    def param_spec(shape):
        return pl.BlockSpec(shape, lambda i: (0, 0))

    in_specs = ([batch_spec(S), batch_spec(S), batch_spec(A)]
                + [param_spec(tuple(p.shape)) for p in param_arrays])

    H1 = w1p.shape[1]
    H2 = w2p.shape[1]
    flops = 2 * b_pad * ((S + A) * H1 + 2 * S * H1 + 2 * H1 * H2
                         + H2 * S + H2 * A)
    bytes_accessed = (4 * b_pad * (2 * S + A + S + A)
                      + 4 * sum(int(p.size) for p in param_arrays))

    ns_pred, a_pred = pl.pallas_call(
        _make_icm_kernel(S),
        out_shape=(jax.ShapeDtypeStruct((b_pad, S), jnp.float32),
                   jax.ShapeDtypeStruct((b_pad, A), jnp.float32)),
        grid=grid,
        in_specs=in_specs,
        out_specs=(pl.BlockSpec((tile_b, S), lambda i: (i, 0)),
                   pl.BlockSpec((tile_b, A), lambda i: (i, 0))),
        compiler_params=pltpu.CompilerParams(
            dimension_semantics=("parallel",)),
        cost_estimate=pl.CostEstimate(
            flops=flops, transcendentals=0, bytes_accessed=bytes_accessed),
    )(state, next_state, action, *param_arrays)

    if b_pad != B:
        ns_pred, a_pred = ns_pred[:B], a_pred[:B]
    return ns_pred, a_pred
